# baseline (speedup 1.0000x reference)
import jax
import jax.numpy as jnp
from jax import lax
from jax.experimental import pallas as pl
from jax.experimental.pallas import tpu as pltpu
from jax.experimental.pallas import tpu_sc as plsc

N = 10000
D_IN = 128
D_H = 256
D_OUT = 128
E = 320000
EL = 20000

NC = 2
NS = 16
W = 128
NB = 2

EP = 327680
NWIN = EP // W
NP = 10240
PAD_ROW = N
RPS = NP // NS
NWIN_HALF = NWIN // NS
NWIN_FULL = NWIN // (NC * NS)
ELP = 40960
NWIN_DEC = ELP // W // (NC * NS)

ROW_BLK = 2000

_MESH = plsc.VectorSubcoreMesh(core_axis_name="c", subcore_axis_name="s")
_f32 = jnp.float32

_AGG_SCRATCH = (
    [pltpu.VMEM_SHARED((NP, 128), _f32)]
    + [pltpu.VMEM((W, 128), _f32) for _ in range(NB)]
    + [pltpu.VMEM((2, W), jnp.int32) for _ in range(NB)]
    + [pltpu.SemaphoreType.DMA for _ in range(2 * NB)]
)


def _zero_acc(z_hbm, acc, buf, sid):
    @pl.loop(0, RPS // W)
    def _(k):
        r = sid * RPS + k * W
        pltpu.sync_copy(z_hbm.at[pl.ds(r, W)], buf)
        pltpu.sync_copy(buf, acc.at[pl.ds(r, W)])


def _dump_acc(acc, out_hbm, buf, sid):
    @pl.loop(0, RPS // W)
    def _(k):
        r = sid * RPS + k * W
        pltpu.sync_copy(acc.at[pl.ds(r, W)], buf)
        pltpu.sync_copy(buf, out_hbm.at[pl.ds(r, W)])


def _agg_run(tab_hbm, out_hbm, idx_hbm, z_hbm, acc, rows, ibufs, gsems,
             ssems, sid, gbase, nwin):
    _zero_acc(z_hbm, acc, rows[0], sid)
    plsc.subcore_barrier()

    def wait_gather(b):
        pltpu.make_async_copy(tab_hbm.at[ibufs[b].at[0]], rows[b], gsems[b]).wait()

    def start_scatter(b):
        pltpu.async_copy(rows[b], acc.at[ibufs[b].at[1]], ssems[b], add=True)

    def wait_scatter(b):
        pltpu.make_async_copy(rows[b], acc.at[ibufs[b].at[1]], ssems[b]).wait()

    def fetch_and_gather(b, win):
        pltpu.sync_copy(idx_hbm.at[win], ibufs[b])
        pltpu.async_copy(tab_hbm.at[ibufs[b].at[0]], rows[b], gsems[b])

    for b in range(NB):
        fetch_and_gather(b, gbase + b)

    @pl.loop(1, nwin // NB)
    def _(g):
        for b in range(NB):
            wait_gather(b)
            start_scatter(b)
        for b in range(NB):
            wait_scatter(b)
            fetch_and_gather(b, gbase + g * NB + b)

    for b in range(NB):
        wait_gather(b)
        start_scatter(b)
    for b in range(NB):
        wait_scatter(b)

    plsc.subcore_barrier()
    _dump_acc(acc, out_hbm, rows[0], sid)


def _deg_body(idx_hbm, z_hbm, o_hbm, out_hbm, acc, r0, r1, i0, i1, g0, g1,
              s0, s1):
    c = lax.axis_index("c")
    sid = lax.axis_index("s")
    wid = sid * NC + c
    ibufs = [i0, i1]
    ssems = [s0, s1]
    ones = r0
    buf = r1

    pltpu.sync_copy(o_hbm, ones)
    _zero_acc(z_hbm, acc, buf, sid)
    plsc.subcore_barrier()

    gbase = wid * NWIN_FULL

    def start_scatter(b):
        pltpu.async_copy(ones, acc.at[ibufs[b].at[1]], ssems[b], add=True)

    def wait_scatter(b):
        pltpu.make_async_copy(ones, acc.at[ibufs[b].at[1]], ssems[b]).wait()

    for b in range(NB):
        pltpu.sync_copy(idx_hbm.at[gbase + b], ibufs[b])
        start_scatter(b)

    @pl.loop(1, NWIN_FULL // NB)
    def _(g):
        for b in range(NB):
            wait_scatter(b)
            pltpu.sync_copy(idx_hbm.at[gbase + g * NB + b], ibufs[b])
            start_scatter(b)

    for b in range(NB):
        wait_scatter(b)

    plsc.subcore_barrier()

    @pl.loop(0, RPS // W)
    def _(k):
        r = sid * RPS + k * W
        pltpu.sync_copy(acc.at[pl.ds(r, W)], buf)
        pltpu.sync_copy(buf, out_hbm.at[c, pl.ds(r, W)])


_deg_call = pl.kernel(
    _deg_body,
    out_type=jax.ShapeDtypeStruct((NC, NP, 128), _f32),
    mesh=_MESH,
    scratch_types=_AGG_SCRATCH,
)


def _agg1_body(tab_a, tab_b, idx_hbm, z_hbm, out_a, out_b, acc, r0, r1,
               i0, i1, g0, g1, s0, s1):
    c = lax.axis_index("c")
    sid = lax.axis_index("s")
    rows = [r0, r1]
    ibufs = [i0, i1]
    gsems = [g0, g1]
    ssems = [s0, s1]
    gbase = sid * NWIN_HALF

    @pl.when(c == 0)
    def _():
        _agg_run(tab_a, out_a, idx_hbm, z_hbm, acc, rows, ibufs, gsems,
                 ssems, sid, gbase, NWIN_HALF)

    @pl.when(c == 1)
    def _():
        _agg_run(tab_b, out_b, idx_hbm, z_hbm, acc, rows, ibufs, gsems,
                 ssems, sid, gbase, NWIN_HALF)


_agg1_call = pl.kernel(
    _agg1_body,
    out_type=[
        jax.ShapeDtypeStruct((NP, 128), _f32),
        jax.ShapeDtypeStruct((NP, 128), _f32),
    ],
    mesh=_MESH,
    scratch_types=_AGG_SCRATCH,
)


def _agg2_body(tab_hbm, idx_hbm, z_hbm, out_hbm, acc, r0, r1, i0, i1,
               g0, g1, s0, s1):
    c = lax.axis_index("c")
    sid = lax.axis_index("s")
    wid = sid * NC + c
    rows = [r0, r1]
    ibufs = [i0, i1]
    gsems = [g0, g1]
    ssems = [s0, s1]

    _agg_run(tab_hbm, out_hbm.at[c], idx_hbm, z_hbm, acc, rows, ibufs,
             gsems, ssems, sid, wid * NWIN_FULL, NWIN_FULL)


_agg2_call = pl.kernel(
    _agg2_body,
    out_type=jax.ShapeDtypeStruct((NC, NP, 128), _f32),
    mesh=_MESH,
    scratch_types=_AGG_SCRATCH,
)


def _dec_body(tab_hbm, idx_hbm, out_hbm, r0, r1, i0, i1, g0, g1, s0, s1):
    c = lax.axis_index("c")
    sid = lax.axis_index("s")
    wid = sid * NC + c
    rows = [r0, r1]
    ibufs = [i0, i1]
    gsems = [g0, g1]
    wsems = [s0, s1]
    gbase = wid * NWIN_DEC

    def fetch_and_gather(b, win):
        pltpu.sync_copy(idx_hbm.at[win], ibufs[b].at[0])
        pltpu.async_copy(tab_hbm.at[ibufs[b].at[0]], rows[b], gsems[b])

    def drain_to_out(b, win):
        pltpu.make_async_copy(tab_hbm.at[ibufs[b].at[0]], rows[b], gsems[b]).wait()
        pltpu.async_copy(rows[b], out_hbm.at[pl.ds(win * W, W)], wsems[b])

    def wait_out(b, win):
        pltpu.make_async_copy(rows[b], out_hbm.at[pl.ds(win * W, W)], wsems[b]).wait()

    for b in range(2):
        fetch_and_gather(b, gbase + b)

    @pl.loop(1, NWIN_DEC // 2)
    def _(g):
        for b in range(2):
            drain_to_out(b, gbase + (g - 1) * 2 + b)
        for b in range(2):
            wait_out(b, gbase + (g - 1) * 2 + b)
            fetch_and_gather(b, gbase + g * 2 + b)

    for b in range(2):
        drain_to_out(b, gbase + NWIN_DEC - 2 + b)
    for b in range(2):
        wait_out(b, gbase + NWIN_DEC - 2 + b)


_dec_call = pl.kernel(
    _dec_body,
    out_type=jax.ShapeDtypeStruct((ELP, 128), _f32),
    mesh=_MESH,
    scratch_types=(
        [pltpu.VMEM((W, 128), _f32) for _ in range(2)]
        + [pltpu.VMEM((1, W), jnp.int32) for _ in range(2)]
        + [pltpu.SemaphoreType.DMA for _ in range(4)]
    ),
)


def _mm_body(x_ref, w_ref, o_ref):
    o_ref[...] = jnp.dot(x_ref[...], w_ref[...], preferred_element_type=jnp.float32)


def _tc_matmul(x, w):
    n, k = x.shape
    m = w.shape[1]
    return pl.pallas_call(
        _mm_body,
        grid=(n // ROW_BLK,),
        in_specs=[
            pl.BlockSpec((ROW_BLK, k), lambda i: (i, 0)),
            pl.BlockSpec((k, m), lambda i: (0, 0)),
        ],
        out_specs=pl.BlockSpec((ROW_BLK, m), lambda i: (i, 0)),
        out_shape=jax.ShapeDtypeStruct((n, m), jnp.float32),
    )(x, w)


def _scale_body(h_ref, dega_ref, degb_ref, hs_ref, dinv_ref):
    dinv = jax.lax.rsqrt(dega_ref[...] + degb_ref[...])
    dinv_ref[...] = dinv
    hs_ref[...] = h_ref[...] * dinv


def _tc_scale(h, dega, degb):
    n, m = h.shape
    return pl.pallas_call(
        _scale_body,
        grid=(n // ROW_BLK,),
        in_specs=[
            pl.BlockSpec((ROW_BLK, m), lambda i: (i, 0)),
            pl.BlockSpec((ROW_BLK, 1), lambda i: (i, 0)),
            pl.BlockSpec((ROW_BLK, 1), lambda i: (i, 0)),
        ],
        out_specs=[
            pl.BlockSpec((ROW_BLK, m), lambda i: (i, 0)),
            pl.BlockSpec((ROW_BLK, 1), lambda i: (i, 0)),
        ],
        out_shape=[
            jax.ShapeDtypeStruct((n, m), jnp.float32),
            jax.ShapeDtypeStruct((n, 1), jnp.float32),
        ],
    )(h, dega, degb)


def _mid_body(agga_ref, aggb_ref, hs1_ref, dinv_ref, b1_ref, w2_ref, hs2_ref):
    agg = jnp.concatenate([agga_ref[...], aggb_ref[...]], axis=-1)
    out1 = jax.nn.relu(dinv_ref[...] * (agg + hs1_ref[...]) + b1_ref[...])
    h2 = jnp.dot(out1, w2_ref[...], preferred_element_type=jnp.float32)
    hs2_ref[...] = h2 * dinv_ref[...]


def _tc_mid(agga, aggb, hs1, dinv, b1, W2):
    n = agga.shape[0]
    return pl.pallas_call(
        _mid_body,
        grid=(n // ROW_BLK,),
        in_specs=[
            pl.BlockSpec((ROW_BLK, 128), lambda i: (i, 0)),
            pl.BlockSpec((ROW_BLK, 128), lambda i: (i, 0)),
            pl.BlockSpec((ROW_BLK, D_H), lambda i: (i, 0)),
            pl.BlockSpec((ROW_BLK, 1), lambda i: (i, 0)),
            pl.BlockSpec((1, D_H), lambda i: (0, 0)),
            pl.BlockSpec((D_H, D_OUT), lambda i: (0, 0)),
        ],
        out_specs=pl.BlockSpec((ROW_BLK, D_OUT), lambda i: (i, 0)),
        out_shape=jax.ShapeDtypeStruct((n, D_OUT), jnp.float32),
    )(agga, aggb, hs1, dinv, b1, W2)


def _z_body(p0_ref, p1_ref, hs2_ref, dinv_ref, b2_ref, z_ref):
    z_ref[...] = (dinv_ref[...] * (p0_ref[...] + p1_ref[...] + hs2_ref[...])
                  + b2_ref[...])


def _tc_z(p0, p1, hs2, dinv, b2):
    n = p0.shape[0]
    return pl.pallas_call(
        _z_body,
        grid=(n // ROW_BLK,),
        in_specs=[
            pl.BlockSpec((ROW_BLK, D_OUT), lambda i: (i, 0)),
            pl.BlockSpec((ROW_BLK, D_OUT), lambda i: (i, 0)),
            pl.BlockSpec((ROW_BLK, D_OUT), lambda i: (i, 0)),
            pl.BlockSpec((ROW_BLK, 1), lambda i: (i, 0)),
            pl.BlockSpec((1, D_OUT), lambda i: (0, 0)),
        ],
        out_specs=pl.BlockSpec((ROW_BLK, D_OUT), lambda i: (i, 0)),
        out_shape=jax.ShapeDtypeStruct((n, D_OUT), jnp.float32),
    )(p0, p1, hs2, dinv, b2)


def _dot_body(zs_ref, zd_ref, o_ref):
    o_ref[...] = jnp.sum(zs_ref[...] * zd_ref[...], axis=-1, keepdims=True)


def _tc_dot(zs, zd):
    n = zs.shape[0]
    return pl.pallas_call(
        _dot_body,
        grid=(n // ROW_BLK,),
        in_specs=[
            pl.BlockSpec((ROW_BLK, D_OUT), lambda i: (i, 0)),
            pl.BlockSpec((ROW_BLK, D_OUT), lambda i: (i, 0)),
        ],
        out_specs=pl.BlockSpec((ROW_BLK, 1), lambda i: (i, 0)),
        out_shape=jax.ShapeDtypeStruct((n, 1), jnp.float32),
    )(zs, zd)


def _pad_rows(a):
    return jnp.concatenate(
        [a, jnp.zeros((NP - a.shape[0], a.shape[1]), a.dtype)], axis=0)


def kernel(x, edge_index, edge_label_index, W1, b1, W2, b2):
    epad = jnp.full((1, EP - E), PAD_ROW, jnp.int32)
    src_p = jnp.concatenate([edge_index[:1], epad], axis=1)
    dst_p = jnp.concatenate([edge_index[1:2], epad], axis=1)
    idx2 = jnp.concatenate(
        [src_p.reshape(NWIN, 1, W), dst_p.reshape(NWIN, 1, W)], axis=1)
    z128 = jnp.zeros((NP, 128), jnp.float32)
    ones = jnp.ones((W, 128), jnp.float32)

    degacc = _deg_call(idx2, z128, ones)
    dega = degacc[0, :N, :1] + 1.0
    degb = degacc[1, :N, :1]

    h1 = _tc_matmul(x, W1)
    hs1, dinv = _tc_scale(h1, dega, degb)

    hs1p = _pad_rows(hs1[:, :128])
    hs1q = _pad_rows(hs1[:, 128:])
    agg_a, agg_b = _agg1_call(hs1p, hs1q, idx2, z128)

    hs2 = _tc_mid(agg_a[:N], agg_b[:N], hs1, dinv, b1[None, :], W2)

    hs2p = _pad_rows(hs2)
    agg2 = _agg2_call(hs2p, idx2, z128)

    z = _tc_z(agg2[0, :N], agg2[1, :N], hs2, dinv, b2[None, :])

    zp = _pad_rows(z)
    lpad = jnp.full((ELP - 2 * EL,), PAD_ROW, jnp.int32)
    dec_idx = jnp.concatenate(
        [edge_label_index[0], edge_label_index[1], lpad]).reshape(ELP // W, W)
    rows = _dec_call(zp, dec_idx)

    return _tc_dot(rows[:EL], rows[EL:2 * EL])[:, 0]
